# baseline (speedup 1.0000x reference)
import jax
import jax.numpy as jnp
from jax.experimental import pallas as pl


def _smooth_l1(d):
    ad = jnp.abs(d)
    return jnp.where(ad < 1.0, 0.5 * ad * ad, ad - 0.5)


def _match_body(t_ref, tt_ref, lp_ref, cp_ref, db_ref, lossc_ref, stats_ref):
    G = t_ref.shape[1]
    N, C = cp_ref.shape[1], cp_ref.shape[2]
    t = t_ref[0]
    tt = tt_ref[0]
    lp = jnp.transpose(lp_ref[0], (1, 0))
    cp = jnp.transpose(cp_ref[0], (1, 0))
    db = db_ref[...]

    gx1 = t[:, 0:1]
    gy1 = t[:, 1:2]
    gx2 = t[:, 2:3]
    gy2 = t[:, 3:4]
    lab = t[:, 4:5]

    pcx = db[0:1]
    pcy = db[1:2]
    pw = db[2:3]
    ph = db[3:4]
    px1 = pcx - pw / 2
    py1 = pcy - ph / 2
    px2 = pcx + pw / 2
    py2 = pcy + ph / 2

    iw = jnp.maximum(jnp.minimum(gx2, px2) - jnp.maximum(gx1, px1), 0.0)
    ih = jnp.maximum(jnp.minimum(gy2, py2) - jnp.maximum(gy1, py1), 0.0)
    inter = iw * ih
    area_a = (gx2 - gx1) * (gy2 - gy1)
    area_b = (px2 - px1) * (py2 - py1)
    iou = inter / (area_a + area_b - inter)

    cidx = jax.lax.broadcasted_iota(jnp.int32, (G, N), 1)
    ridx = jax.lax.broadcasted_iota(jnp.int32, (G, N), 0)

    gbo = jnp.max(iou, axis=1, keepdims=True)
    gbi = jnp.min(jnp.where(iou == gbo, cidx, N), axis=1, keepdims=True)
    valid = gbo >= 0.2
    hv = jnp.max(valid.astype(jnp.float32), axis=0, keepdims=True)

    dbo = jnp.max(iou, axis=0, keepdims=True)

    pow2_row = jax.lax.bitcast_convert_type(
        (jax.lax.broadcasted_iota(jnp.int32, (1, G), 1) + 127) << 23,
        jnp.float32)
    pow2_col = jax.lax.bitcast_convert_type(
        (jax.lax.broadcasted_iota(jnp.int32, (G, 1), 0) + 127) << 23,
        jnp.float32)
    dn = (((1,), (0,)), ((), ()))
    hi = jax.lax.Precision.HIGHEST

    mm = gbi == cidx
    eq = iou == dbo
    fbits_f = jax.lax.dot_general(pow2_row, mm.astype(jnp.float32), dn)
    dbits_f = jax.lax.dot_general(pow2_row, eq.astype(jnp.float32), dn)
    fbits = fbits_f.astype(jnp.int32)
    vbits = jnp.sum(pow2_col * valid.astype(jnp.float32), axis=0,
                    keepdims=True).astype(jnp.int32)
    set_mask = (fbits & vbits) != 0
    force = (jax.lax.bitcast_convert_type(fbits_f, jnp.int32) >> 23) - 127
    dbits = dbits_f.astype(jnp.int32)
    low_f = (dbits & (-dbits)).astype(jnp.float32)
    dbi = (jax.lax.bitcast_convert_type(low_f, jnp.int32) >> 23) - 127
    didx = jnp.where(fbits > 0, force, dbi)
    ovl = jnp.where(set_mask, 1.0, dbo)

    oh = didx == ridx
    d1 = jax.lax.dot_general(tt, oh.astype(jnp.float32), dn,
                             precision=hi)
    mx1 = d1[0:1]
    my1 = d1[1:2]
    mx2 = d1[2:3]
    my2 = d1[3:4]
    conf = d1[4:5]

    conf = jnp.where(ovl < 0.5, 0.0, conf)
    conf = jnp.where(hv > 0.0, conf, 0.0)
    pos = conf > 0.0

    g_cx = ((mx1 + mx2) / 2 - pcx) / (0.1 * pw)
    g_cy = ((my1 + my2) / 2 - pcy) / (0.1 * ph)
    g_w = jnp.log((mx2 - mx1) / pw) / 0.2
    g_h = jnp.log((my2 - my1) / ph) / 0.2

    sl = (jnp.where(pos, _smooth_l1(lp[0:1] - g_cx), 0.0)
          + jnp.where(pos, _smooth_l1(lp[1:2] - g_cy), 0.0)
          + jnp.where(pos, _smooth_l1(lp[2:3] - g_w), 0.0)
          + jnp.where(pos, _smooth_l1(lp[3:4] - g_h), 0.0))
    lloc = jnp.sum(sl, axis=1, keepdims=True)

    m = jnp.max(cp, axis=0, keepdims=True)
    lse = jnp.log(jnp.sum(jnp.exp(cp - m), axis=0, keepdims=True)) + m
    klass = jax.lax.broadcasted_iota(jnp.int32, (C, N), 0)
    conf_i = conf.astype(jnp.int32)
    gathered = jnp.sum(jnp.where(klass == conf_i, cp, 0.0), axis=0,
                       keepdims=True)
    ce = lse - gathered

    sum_ce_pos = jnp.sum(jnp.where(pos, ce, 0.0), axis=1, keepdims=True)
    npos = jnp.sum(pos.astype(jnp.float32), axis=1, keepdims=True)

    lossc_ref[0] = jnp.maximum(jnp.where(pos, 0.0, ce), 0.0)

    lane = jax.lax.broadcasted_iota(jnp.int32, (1, 128), 1)
    svec = jnp.where(lane == 0, lloc,
                     jnp.where(lane == 1, sum_ce_pos,
                               jnp.where(lane == 2, npos, 0.0)))
    stats_ref[0] = svec


def _select_body(lossc_ref, stats_ref, out0_ref, out1_ref):
    B, N = lossc_ref.shape
    lc = lossc_ref[...]
    st = stats_ref[...]
    npos = st[:, 2:3]
    k = jnp.minimum(3 * npos.astype(jnp.int32), N - 2) + 1

    bits = jax.lax.bitcast_convert_type(lc, jnp.int32)
    t = jnp.zeros((B, 1), jnp.int32)
    for bit in range(30, -1, -1):
        cand = t | (1 << bit)
        cnt = jnp.sum((bits >= cand).astype(jnp.int32), axis=1,
                      keepdims=True)
        t = jnp.where(cnt >= k, cand, t)

    tf = jax.lax.bitcast_convert_type(t, jnp.float32)
    gtm = bits > t
    cnt_gt = jnp.sum(gtm.astype(jnp.int32), axis=1, keepdims=True)
    sum_gt = jnp.sum(jnp.where(gtm, lc, 0.0), axis=1, keepdims=True)
    topk = sum_gt + (k - cnt_gt).astype(jnp.float32) * tf

    loss_conf = (jnp.sum(st[:, 1:2], axis=0, keepdims=True)
                 + jnp.sum(topk, axis=0, keepdims=True))
    loss_loc = jnp.sum(st[:, 0:1], axis=0, keepdims=True)
    nn = jnp.maximum(jnp.sum(npos, axis=0, keepdims=True), 1.0)
    out0_ref[...] = loss_loc / nn
    out1_ref[...] = loss_conf / nn


def kernel(loc_p, conf_p, targets, default_boxes):
    B, N, _ = loc_p.shape
    C = conf_p.shape[2]
    G = targets.shape[1]

    targets_t = jnp.transpose(targets, (0, 2, 1))
    db_t = jnp.transpose(default_boxes, (1, 0))

    lossc, stats = pl.pallas_call(
        _match_body,
        grid=(B,),
        in_specs=[
            pl.BlockSpec((1, G, 5), lambda b: (b, 0, 0)),
            pl.BlockSpec((1, 5, G), lambda b: (b, 0, 0)),
            pl.BlockSpec((1, N, 4), lambda b: (b, 0, 0)),
            pl.BlockSpec((1, N, C), lambda b: (b, 0, 0)),
            pl.BlockSpec((4, N), lambda b: (0, 0)),
        ],
        out_specs=[
            pl.BlockSpec((1, 1, N), lambda b: (b, 0, 0)),
            pl.BlockSpec((1, 1, 128), lambda b: (b, 0, 0)),
        ],
        out_shape=[
            jax.ShapeDtypeStruct((B, 1, N), jnp.float32),
            jax.ShapeDtypeStruct((B, 1, 128), jnp.float32),
        ],
    )(targets, targets_t, loc_p, conf_p, db_t)

    out0, out1 = pl.pallas_call(
        _select_body,
        out_shape=[
            jax.ShapeDtypeStruct((1, 1), jnp.float32),
            jax.ShapeDtypeStruct((1, 1), jnp.float32),
        ],
    )(lossc.reshape(B, N), stats.reshape(B, 128))

    return (out0.reshape(()), out1.reshape(()))


# speedup vs baseline: 1.9153x; 1.9153x over previous
import jax
import jax.numpy as jnp
from jax.experimental import pallas as pl


def _smooth_l1(d):
    ad = jnp.abs(d)
    return jnp.where(ad < 1.0, 0.5 * ad * ad, ad - 0.5)


def _match_body(t_ref, tt_ref, lp_ref, cp_ref, db_ref, lossc_ref, stats_ref):
    G = t_ref.shape[1]
    C, N = cp_ref.shape[1], cp_ref.shape[2]
    t = t_ref[0]
    tt = tt_ref[0]
    lp = lp_ref[0]
    cp = cp_ref[0]
    db = db_ref[...]

    gx1 = t[:, 0:1]
    gy1 = t[:, 1:2]
    gx2 = t[:, 2:3]
    gy2 = t[:, 3:4]
    lab = t[:, 4:5]

    pcx = db[0:1]
    pcy = db[1:2]
    pw = db[2:3]
    ph = db[3:4]
    px1 = pcx - pw / 2
    py1 = pcy - ph / 2
    px2 = pcx + pw / 2
    py2 = pcy + ph / 2

    iw = jnp.maximum(jnp.minimum(gx2, px2) - jnp.maximum(gx1, px1), 0.0)
    ih = jnp.maximum(jnp.minimum(gy2, py2) - jnp.maximum(gy1, py1), 0.0)
    inter = iw * ih
    area_a = (gx2 - gx1) * (gy2 - gy1)
    area_b = (px2 - px1) * (py2 - py1)
    iou = inter / (area_a + area_b - inter)

    cidx = jax.lax.broadcasted_iota(jnp.int32, (G, N), 1)
    ridx = jax.lax.broadcasted_iota(jnp.int32, (G, N), 0)

    gbo = jnp.max(iou, axis=1, keepdims=True)
    gbi = jnp.min(jnp.where(iou == gbo, cidx, N), axis=1, keepdims=True)
    valid = gbo >= 0.2
    hv = jnp.max(valid.astype(jnp.float32), axis=0, keepdims=True)

    dbo = jnp.max(iou, axis=0, keepdims=True)

    pow2_row = jax.lax.bitcast_convert_type(
        (jax.lax.broadcasted_iota(jnp.int32, (1, G), 1) + 127) << 23,
        jnp.float32)
    pow2_col = jax.lax.bitcast_convert_type(
        (jax.lax.broadcasted_iota(jnp.int32, (G, 1), 0) + 127) << 23,
        jnp.float32)
    dn = (((1,), (0,)), ((), ()))
    hi = jax.lax.Precision.HIGHEST

    mm = gbi == cidx
    eq = iou == dbo
    fbits_f = jax.lax.dot_general(pow2_row, mm.astype(jnp.float32), dn)
    dbits_f = jax.lax.dot_general(pow2_row, eq.astype(jnp.float32), dn)
    fbits = fbits_f.astype(jnp.int32)
    vbits = jnp.sum(pow2_col * valid.astype(jnp.float32), axis=0,
                    keepdims=True).astype(jnp.int32)
    set_mask = (fbits & vbits) != 0
    force = (jax.lax.bitcast_convert_type(fbits_f, jnp.int32) >> 23) - 127
    dbits = dbits_f.astype(jnp.int32)
    low_f = (dbits & (-dbits)).astype(jnp.float32)
    dbi = (jax.lax.bitcast_convert_type(low_f, jnp.int32) >> 23) - 127
    didx = jnp.where(fbits > 0, force, dbi)
    ovl = jnp.where(set_mask, 1.0, dbo)

    oh = didx == ridx
    d1 = jax.lax.dot_general(tt, oh.astype(jnp.float32), dn,
                             precision=hi)
    mx1 = d1[0:1]
    my1 = d1[1:2]
    mx2 = d1[2:3]
    my2 = d1[3:4]
    conf = d1[4:5]

    conf = jnp.where(ovl < 0.5, 0.0, conf)
    conf = jnp.where(hv > 0.0, conf, 0.0)
    pos = conf > 0.0

    g_cx = ((mx1 + mx2) / 2 - pcx) / (0.1 * pw)
    g_cy = ((my1 + my2) / 2 - pcy) / (0.1 * ph)
    g_w = jnp.log((mx2 - mx1) / pw) / 0.2
    g_h = jnp.log((my2 - my1) / ph) / 0.2

    sl = (jnp.where(pos, _smooth_l1(lp[0:1] - g_cx), 0.0)
          + jnp.where(pos, _smooth_l1(lp[1:2] - g_cy), 0.0)
          + jnp.where(pos, _smooth_l1(lp[2:3] - g_w), 0.0)
          + jnp.where(pos, _smooth_l1(lp[3:4] - g_h), 0.0))
    lloc = jnp.sum(sl, axis=1, keepdims=True)

    m = jnp.max(cp, axis=0, keepdims=True)
    lse = jnp.log(jnp.sum(jnp.exp(cp - m), axis=0, keepdims=True)) + m
    klass = jax.lax.broadcasted_iota(jnp.int32, (C, N), 0)
    conf_i = conf.astype(jnp.int32)
    gathered = jnp.sum(jnp.where(klass == conf_i, cp, 0.0), axis=0,
                       keepdims=True)
    ce = lse - gathered

    sum_ce_pos = jnp.sum(jnp.where(pos, ce, 0.0), axis=1, keepdims=True)
    npos = jnp.sum(pos.astype(jnp.float32), axis=1, keepdims=True)

    lossc_ref[0] = jnp.maximum(jnp.where(pos, 0.0, ce), 0.0)

    lane = jax.lax.broadcasted_iota(jnp.int32, (1, 128), 1)
    svec = jnp.where(lane == 0, lloc,
                     jnp.where(lane == 1, sum_ce_pos,
                               jnp.where(lane == 2, npos, 0.0)))
    stats_ref[0] = svec


def _select_body(lossc_ref, stats_ref, out0_ref, out1_ref):
    B, N = lossc_ref.shape
    lc = lossc_ref[...]
    st = stats_ref[...]
    npos = st[:, 2:3]
    k = jnp.minimum(3 * npos.astype(jnp.int32), N - 2) + 1

    bits = jax.lax.bitcast_convert_type(lc, jnp.int32)
    t = jnp.zeros((B, 1), jnp.int32)
    for bit in range(30, -1, -1):
        cand = t | (1 << bit)
        cnt = jnp.sum((bits >= cand).astype(jnp.int32), axis=1,
                      keepdims=True)
        t = jnp.where(cnt >= k, cand, t)

    tf = jax.lax.bitcast_convert_type(t, jnp.float32)
    gtm = bits > t
    cnt_gt = jnp.sum(gtm.astype(jnp.int32), axis=1, keepdims=True)
    sum_gt = jnp.sum(jnp.where(gtm, lc, 0.0), axis=1, keepdims=True)
    topk = sum_gt + (k - cnt_gt).astype(jnp.float32) * tf

    loss_conf = (jnp.sum(st[:, 1:2], axis=0, keepdims=True)
                 + jnp.sum(topk, axis=0, keepdims=True))
    loss_loc = jnp.sum(st[:, 0:1], axis=0, keepdims=True)
    nn = jnp.maximum(jnp.sum(npos, axis=0, keepdims=True), 1.0)
    out0_ref[...] = loss_loc / nn
    out1_ref[...] = loss_conf / nn


def kernel(loc_p, conf_p, targets, default_boxes):
    B, N, _ = loc_p.shape
    C = conf_p.shape[2]
    G = targets.shape[1]

    loc_pt = jnp.transpose(loc_p, (0, 2, 1))
    conf_pt = jnp.transpose(conf_p, (0, 2, 1))
    targets_t = jnp.transpose(targets, (0, 2, 1))
    db_t = jnp.transpose(default_boxes, (1, 0))

    lossc, stats = pl.pallas_call(
        _match_body,
        grid=(B,),
        in_specs=[
            pl.BlockSpec((1, G, 5), lambda b: (b, 0, 0)),
            pl.BlockSpec((1, 5, G), lambda b: (b, 0, 0)),
            pl.BlockSpec((1, 4, N), lambda b: (b, 0, 0)),
            pl.BlockSpec((1, C, N), lambda b: (b, 0, 0)),
            pl.BlockSpec((4, N), lambda b: (0, 0)),
        ],
        out_specs=[
            pl.BlockSpec((1, 1, N), lambda b: (b, 0, 0)),
            pl.BlockSpec((1, 1, 128), lambda b: (b, 0, 0)),
        ],
        out_shape=[
            jax.ShapeDtypeStruct((B, 1, N), jnp.float32),
            jax.ShapeDtypeStruct((B, 1, 128), jnp.float32),
        ],
    )(targets, targets_t, loc_pt, conf_pt, db_t)

    out0, out1 = pl.pallas_call(
        _select_body,
        out_shape=[
            jax.ShapeDtypeStruct((1, 1), jnp.float32),
            jax.ShapeDtypeStruct((1, 1), jnp.float32),
        ],
    )(lossc.reshape(B, N), stats.reshape(B, 128))

    return (out0.reshape(()), out1.reshape(()))


# speedup vs baseline: 2.0344x; 1.0622x over previous
import jax
import jax.numpy as jnp
from jax.experimental import pallas as pl


def _smooth_l1(d):
    ad = jnp.abs(d)
    return jnp.where(ad < 1.0, 0.5 * ad * ad, ad - 0.5)


def _match_body(t_ref, tt_ref, db_ref, enc_ref):
    G = t_ref.shape[1]
    N = db_ref.shape[1]
    t = t_ref[0]
    tt = tt_ref[0]
    db = db_ref[...]

    gx1 = t[:, 0:1]
    gy1 = t[:, 1:2]
    gx2 = t[:, 2:3]
    gy2 = t[:, 3:4]
    lab = t[:, 4:5]

    pcx = db[0:1]
    pcy = db[1:2]
    pw = db[2:3]
    ph = db[3:4]
    px1 = pcx - pw / 2
    py1 = pcy - ph / 2
    px2 = pcx + pw / 2
    py2 = pcy + ph / 2

    iw = jnp.maximum(jnp.minimum(gx2, px2) - jnp.maximum(gx1, px1), 0.0)
    ih = jnp.maximum(jnp.minimum(gy2, py2) - jnp.maximum(gy1, py1), 0.0)
    inter = iw * ih
    area_a = (gx2 - gx1) * (gy2 - gy1)
    area_b = (px2 - px1) * (py2 - py1)
    iou = inter / (area_a + area_b - inter)

    cidx = jax.lax.broadcasted_iota(jnp.int32, (G, N), 1)
    ridx = jax.lax.broadcasted_iota(jnp.int32, (G, N), 0)

    gbo = jnp.max(iou, axis=1, keepdims=True)
    gbi = jnp.min(jnp.where(iou == gbo, cidx, N), axis=1, keepdims=True)
    valid = gbo >= 0.2
    hv = jnp.max(valid.astype(jnp.float32), axis=0, keepdims=True)

    dbo = jnp.max(iou, axis=0, keepdims=True)

    pow2_row = jax.lax.bitcast_convert_type(
        (jax.lax.broadcasted_iota(jnp.int32, (1, G), 1) + 127) << 23,
        jnp.float32)
    pow2_col = jax.lax.bitcast_convert_type(
        (jax.lax.broadcasted_iota(jnp.int32, (G, 1), 0) + 127) << 23,
        jnp.float32)
    dn = (((1,), (0,)), ((), ()))
    hi = jax.lax.Precision.HIGHEST

    mm = gbi == cidx
    eq = iou == dbo
    fbits_f = jax.lax.dot_general(pow2_row, mm.astype(jnp.float32), dn)
    dbits_f = jax.lax.dot_general(pow2_row, eq.astype(jnp.float32), dn)
    fbits = fbits_f.astype(jnp.int32)
    vbits = jnp.sum(pow2_col * valid.astype(jnp.float32), axis=0,
                    keepdims=True).astype(jnp.int32)
    set_mask = (fbits & vbits) != 0
    force = (jax.lax.bitcast_convert_type(fbits_f, jnp.int32) >> 23) - 127
    dbits = dbits_f.astype(jnp.int32)
    low_f = (dbits & (-dbits)).astype(jnp.float32)
    dbi = (jax.lax.bitcast_convert_type(low_f, jnp.int32) >> 23) - 127
    didx = jnp.where(fbits > 0, force, dbi)
    ovl = jnp.where(set_mask, 1.0, dbo)

    oh = didx == ridx
    d1 = jax.lax.dot_general(tt, oh.astype(jnp.float32), dn,
                             precision=hi)
    mx1 = d1[0:1]
    my1 = d1[1:2]
    mx2 = d1[2:3]
    my2 = d1[3:4]
    conf = d1[4:5]

    conf = jnp.where(ovl < 0.5, 0.0, conf)
    conf = jnp.where(hv > 0.0, conf, 0.0)

    g_cx =((mx1 + mx2) / 2 - pcx) / (0.1 * pw)
    g_cy = ((my1 + my2) / 2 - pcy) / (0.1 * ph)
    g_w = jnp.log((mx2 - mx1) / pw) / 0.2
    g_h = jnp.log((my2 - my1) / ph) / 0.2

    enc_ref[0] = jnp.concatenate(
        [g_cx, g_cy, g_w, g_h, conf, conf, conf, conf], axis=0)


def _loss_body(enc_ref, lp_ref, cp_ref, lossc_ref, stats_ref):
    C, N = cp_ref.shape[1], cp_ref.shape[2]
    e = enc_ref[0]
    lp = lp_ref[0]
    cp = cp_ref[0]
    g_cx = e[0:1]
    g_cy = e[1:2]
    g_w = e[2:3]
    g_h = e[3:4]
    conf = e[4:5]
    pos = conf > 0.0

    sl = (jnp.where(pos, _smooth_l1(lp[0:1] - g_cx), 0.0)
          + jnp.where(pos, _smooth_l1(lp[1:2] - g_cy), 0.0)
          + jnp.where(pos, _smooth_l1(lp[2:3] - g_w), 0.0)
          + jnp.where(pos, _smooth_l1(lp[3:4] - g_h), 0.0))
    lloc = jnp.sum(sl, axis=1, keepdims=True)

    m = jnp.max(cp, axis=0, keepdims=True)
    lse = jnp.log(jnp.sum(jnp.exp(cp - m), axis=0, keepdims=True)) + m
    klass = jax.lax.broadcasted_iota(jnp.int32, (C, N), 0)
    conf_i = conf.astype(jnp.int32)
    gathered = jnp.sum(jnp.where(klass == conf_i, cp, 0.0), axis=0,
                       keepdims=True)
    ce = lse - gathered

    sum_ce_pos = jnp.sum(jnp.where(pos, ce, 0.0), axis=1, keepdims=True)
    npos = jnp.sum(pos.astype(jnp.float32), axis=1, keepdims=True)

    lossc_ref[0] = jnp.maximum(jnp.where(pos, 0.0, ce), 0.0)

    lane = jax.lax.broadcasted_iota(jnp.int32, (1, 128), 1)
    svec = jnp.where(lane == 0, lloc,
                     jnp.where(lane == 1, sum_ce_pos,
                               jnp.where(lane == 2, npos, 0.0)))
    stats_ref[0] = svec


def _select_body(lossc_ref, stats_ref, out0_ref, out1_ref):
    B, N = lossc_ref.shape
    lc = lossc_ref[...]
    st = stats_ref[...]
    npos = st[:, 2:3]
    k = jnp.minimum(3 * npos.astype(jnp.int32), N - 2) + 1

    bits = jax.lax.bitcast_convert_type(lc, jnp.int32)
    t = jnp.zeros((B, 1), jnp.int32)
    for bit in range(30, -1, -1):
        cand = t | (1 << bit)
        cnt = jnp.sum((bits >= cand).astype(jnp.int32), axis=1,
                      keepdims=True)
        t = jnp.where(cnt >= k, cand, t)

    tf = jax.lax.bitcast_convert_type(t, jnp.float32)
    gtm = bits > t
    cnt_gt = jnp.sum(gtm.astype(jnp.int32), axis=1, keepdims=True)
    sum_gt = jnp.sum(jnp.where(gtm, lc, 0.0), axis=1, keepdims=True)
    topk = sum_gt + (k - cnt_gt).astype(jnp.float32) * tf

    loss_conf = (jnp.sum(st[:, 1:2], axis=0, keepdims=True)
                 + jnp.sum(topk, axis=0, keepdims=True))
    loss_loc = jnp.sum(st[:, 0:1], axis=0, keepdims=True)
    nn = jnp.maximum(jnp.sum(npos, axis=0, keepdims=True), 1.0)
    out0_ref[...] = loss_loc / nn
    out1_ref[...] = loss_conf / nn


def kernel(loc_p, conf_p, targets, default_boxes):
    B, N, _ = loc_p.shape
    C = conf_p.shape[2]
    G = targets.shape[1]

    loc_pt = jnp.transpose(loc_p, (0, 2, 1))
    conf_pt = jnp.transpose(conf_p, (0, 2, 1))
    targets_t = jnp.transpose(targets, (0, 2, 1))
    db_t = jnp.transpose(default_boxes, (1, 0))

    enc = pl.pallas_call(
        _match_body,
        grid=(B,),
        in_specs=[
            pl.BlockSpec((1, G, 5), lambda b: (b, 0, 0)),
            pl.BlockSpec((1, 5, G), lambda b: (b, 0, 0)),
            pl.BlockSpec((4, N), lambda b: (0, 0)),
        ],
        out_specs=pl.BlockSpec((1, 8, N), lambda b: (b, 0, 0)),
        out_shape=jax.ShapeDtypeStruct((B, 8, N), jnp.float32),
    )(targets, targets_t, db_t)

    lossc, stats = pl.pallas_call(
        _loss_body,
        grid=(B,),
        in_specs=[
            pl.BlockSpec((1, 8, N), lambda b: (b, 0, 0)),
            pl.BlockSpec((1, 4, N), lambda b: (b, 0, 0)),
            pl.BlockSpec((1, C, N), lambda b: (b, 0, 0)),
        ],
        out_specs=[
            pl.BlockSpec((1, 1, N), lambda b: (b, 0, 0)),
            pl.BlockSpec((1, 1, 128), lambda b: (b, 0, 0)),
        ],
        out_shape=[
            jax.ShapeDtypeStruct((B, 1, N), jnp.float32),
            jax.ShapeDtypeStruct((B, 1, 128), jnp.float32),
        ],
    )(enc, loc_pt, conf_pt)

    out0, out1 = pl.pallas_call(
        _select_body,
        out_shape=[
            jax.ShapeDtypeStruct((1, 1), jnp.float32),
            jax.ShapeDtypeStruct((1, 1), jnp.float32),
        ],
    )(lossc.reshape(B, N), stats.reshape(B, 128))

    return (out0.reshape(()), out1.reshape(()))


# speedup vs baseline: 2.0396x; 1.0026x over previous
import jax
import jax.numpy as jnp
from jax.experimental import pallas as pl


def _smooth_l1(d):
    ad = jnp.abs(d)
    return jnp.where(ad < 1.0, 0.5 * ad * ad, ad - 0.5)


def _match_body(t_ref, tt_ref, db_ref, enc_ref):
    G = t_ref.shape[1]
    N = db_ref.shape[1]
    t = t_ref[0]
    tt = tt_ref[0]
    db = db_ref[...]

    gx1 = t[:, 0:1]
    gy1 = t[:, 1:2]
    gx2 = t[:, 2:3]
    gy2 = t[:, 3:4]
    lab = t[:, 4:5]

    pcx = db[0:1]
    pcy = db[1:2]
    pw = db[2:3]
    ph = db[3:4]
    px1 = pcx - pw / 2
    py1 = pcy - ph / 2
    px2 = pcx + pw / 2
    py2 = pcy + ph / 2

    iw = jnp.maximum(jnp.minimum(gx2, px2) - jnp.maximum(gx1, px1), 0.0)
    ih = jnp.maximum(jnp.minimum(gy2, py2) - jnp.maximum(gy1, py1), 0.0)
    inter = iw * ih
    area_a = (gx2 - gx1) * (gy2 - gy1)
    area_b = (px2 - px1) * (py2 - py1)
    iou = inter / (area_a + area_b - inter)

    cidx = jax.lax.broadcasted_iota(jnp.int32, (G, N), 1)
    ridx = jax.lax.broadcasted_iota(jnp.int32, (G, N), 0)

    gbo = jnp.max(iou, axis=1, keepdims=True)
    gbi = jnp.min(jnp.where(iou == gbo, cidx, N), axis=1, keepdims=True)
    valid = gbo >= 0.2
    hv = jnp.max(valid.astype(jnp.float32), axis=0, keepdims=True)

    dbo = jnp.max(iou, axis=0, keepdims=True)

    pow2_row = jax.lax.bitcast_convert_type(
        (jax.lax.broadcasted_iota(jnp.int32, (1, G), 1) + 127) << 23,
        jnp.float32)
    pow2_col = jax.lax.bitcast_convert_type(
        (jax.lax.broadcasted_iota(jnp.int32, (G, 1), 0) + 127) << 23,
        jnp.float32)
    dn = (((1,), (0,)), ((), ()))
    hi = jax.lax.Precision.HIGHEST

    mm = gbi == cidx
    eq = iou == dbo
    fbits_f = jax.lax.dot_general(pow2_row, mm.astype(jnp.float32), dn)
    dbits_f = jax.lax.dot_general(pow2_row, eq.astype(jnp.float32), dn)
    fbits = fbits_f.astype(jnp.int32)
    vbits = jnp.sum(pow2_col * valid.astype(jnp.float32), axis=0,
                    keepdims=True).astype(jnp.int32)
    set_mask = (fbits & vbits) != 0
    force = (jax.lax.bitcast_convert_type(fbits_f, jnp.int32) >> 23) - 127
    dbits = dbits_f.astype(jnp.int32)
    low_f = (dbits & (-dbits)).astype(jnp.float32)
    dbi = (jax.lax.bitcast_convert_type(low_f, jnp.int32) >> 23) - 127
    didx = jnp.where(fbits > 0, force, dbi)
    ovl = jnp.where(set_mask, 1.0, dbo)

    oh = didx == ridx
    d1 = jax.lax.dot_general(tt, oh.astype(jnp.float32), dn,
                             precision=hi)
    mx1 = d1[0:1]
    my1 = d1[1:2]
    mx2 = d1[2:3]
    my2 = d1[3:4]
    conf = d1[4:5]

    conf = jnp.where(ovl < 0.5, 0.0, conf)
    conf = jnp.where(hv > 0.0, conf, 0.0)

    g_cx =((mx1 + mx2) / 2 - pcx) / (0.1 * pw)
    g_cy = ((my1 + my2) / 2 - pcy) / (0.1 * ph)
    g_w = jnp.log((mx2 - mx1) / pw) / 0.2
    g_h = jnp.log((my2 - my1) / ph) / 0.2

    enc_ref[0] = jnp.concatenate(
        [g_cx, g_cy, g_w, g_h, conf, conf, conf, conf], axis=0)


def _loss_body(enc_ref, lp_ref, cp_ref, lossc_ref, stats_ref):
    C, N = cp_ref.shape[1], cp_ref.shape[2]
    e = enc_ref[0]
    lp = lp_ref[0]
    cp = cp_ref[0]
    conf = e[4:5]
    pos = conf > 0.0

    sl = jnp.where(pos, _smooth_l1(lp - e[0:4]), 0.0)
    lloc = jnp.sum(jnp.sum(sl, axis=0, keepdims=True), axis=1,
                   keepdims=True)

    m = jnp.max(cp, axis=0, keepdims=True)
    lse = jnp.log(jnp.sum(jnp.exp(cp - m), axis=0, keepdims=True)) + m
    klass = jax.lax.broadcasted_iota(jnp.int32, (C, N), 0)
    conf_i = conf.astype(jnp.int32)
    gathered = jnp.sum(jnp.where(klass == conf_i, cp, 0.0), axis=0,
                       keepdims=True)
    ce = lse - gathered

    sum_ce_pos = jnp.sum(jnp.where(pos, ce, 0.0), axis=1, keepdims=True)
    npos = jnp.sum(pos.astype(jnp.float32), axis=1, keepdims=True)

    lossc_ref[0] = jnp.maximum(jnp.where(pos, 0.0, ce), 0.0)

    lane = jax.lax.broadcasted_iota(jnp.int32, (1, 128), 1)
    svec = jnp.where(lane == 0, lloc,
                     jnp.where(lane == 1, sum_ce_pos,
                               jnp.where(lane == 2, npos, 0.0)))
    stats_ref[0] = svec


def _select_body(lossc_ref, stats_ref, out0_ref, out1_ref):
    B, N = lossc_ref.shape
    lc = lossc_ref[...]
    st = stats_ref[...]
    npos = st[:, 2:3]
    k = jnp.minimum(3 * npos.astype(jnp.int32), N - 2) + 1

    bits = jax.lax.bitcast_convert_type(lc, jnp.int32)
    t = jnp.zeros((B, 1), jnp.int32)
    for bit in range(30, -1, -1):
        cand = t | (1 << bit)
        cnt = jnp.sum((bits >= cand).astype(jnp.int32), axis=1,
                      keepdims=True)
        t = jnp.where(cnt >= k, cand, t)

    tf = jax.lax.bitcast_convert_type(t, jnp.float32)
    gtm = bits > t
    cnt_gt = jnp.sum(gtm.astype(jnp.int32), axis=1, keepdims=True)
    sum_gt = jnp.sum(jnp.where(gtm, lc, 0.0), axis=1, keepdims=True)
    topk = sum_gt + (k - cnt_gt).astype(jnp.float32) * tf

    loss_conf = (jnp.sum(st[:, 1:2], axis=0, keepdims=True)
                 + jnp.sum(topk, axis=0, keepdims=True))
    loss_loc = jnp.sum(st[:, 0:1], axis=0, keepdims=True)
    nn = jnp.maximum(jnp.sum(npos, axis=0, keepdims=True), 1.0)
    out0_ref[...] = loss_loc / nn
    out1_ref[...] = loss_conf / nn


def kernel(loc_p, conf_p, targets, default_boxes):
    B, N, _ = loc_p.shape
    C = conf_p.shape[2]
    G = targets.shape[1]

    loc_pt = jnp.transpose(loc_p, (0, 2, 1))
    conf_pt = jnp.transpose(conf_p, (0, 2, 1))
    targets_t = jnp.transpose(targets, (0, 2, 1))
    db_t = jnp.transpose(default_boxes, (1, 0))

    enc = pl.pallas_call(
        _match_body,
        grid=(B,),
        in_specs=[
            pl.BlockSpec((1, G, 5), lambda b: (b, 0, 0)),
            pl.BlockSpec((1, 5, G), lambda b: (b, 0, 0)),
            pl.BlockSpec((4, N), lambda b: (0, 0)),
        ],
        out_specs=pl.BlockSpec((1, 8, N), lambda b: (b, 0, 0)),
        out_shape=jax.ShapeDtypeStruct((B, 8, N), jnp.float32),
    )(targets, targets_t, db_t)

    lossc, stats = pl.pallas_call(
        _loss_body,
        grid=(B,),
        in_specs=[
            pl.BlockSpec((1, 8, N), lambda b: (b, 0, 0)),
            pl.BlockSpec((1, 4, N), lambda b: (b, 0, 0)),
            pl.BlockSpec((1, C, N), lambda b: (b, 0, 0)),
        ],
        out_specs=[
            pl.BlockSpec((1, 1, N), lambda b: (b, 0, 0)),
            pl.BlockSpec((1, 1, 128), lambda b: (b, 0, 0)),
        ],
        out_shape=[
            jax.ShapeDtypeStruct((B, 1, N), jnp.float32),
            jax.ShapeDtypeStruct((B, 1, 128), jnp.float32),
        ],
    )(enc, loc_pt, conf_pt)

    out0, out1 = pl.pallas_call(
        _select_body,
        out_shape=[
            jax.ShapeDtypeStruct((1, 1), jnp.float32),
            jax.ShapeDtypeStruct((1, 1), jnp.float32),
        ],
    )(lossc.reshape(B, N), stats.reshape(B, 128))

    return (out0.reshape(()), out1.reshape(()))


# speedup vs baseline: 2.1565x; 1.0573x over previous
import jax
import jax.numpy as jnp
from jax.experimental import pallas as pl


def _smooth_l1(d):
    ad = jnp.abs(d)
    return jnp.where(ad < 1.0, 0.5 * ad * ad, ad - 0.5)


def _match_body(t_ref, tt_ref, db_ref, enc_ref):
    G = t_ref.shape[1]
    N = db_ref.shape[1]
    t = t_ref[0]
    tt = tt_ref[0]
    db = db_ref[...]

    gx1 = t[:, 0:1]
    gy1 = t[:, 1:2]
    gx2 = t[:, 2:3]
    gy2 = t[:, 3:4]
    lab = t[:, 4:5]

    pcx = db[0:1]
    pcy = db[1:2]
    pw = db[2:3]
    ph = db[3:4]
    px1 = pcx - pw / 2
    py1 = pcy - ph / 2
    px2 = pcx + pw / 2
    py2 = pcy + ph / 2

    iw = jnp.maximum(jnp.minimum(gx2, px2) - jnp.maximum(gx1, px1), 0.0)
    ih = jnp.maximum(jnp.minimum(gy2, py2) - jnp.maximum(gy1, py1), 0.0)
    inter = iw * ih
    area_a = (gx2 - gx1) * (gy2 - gy1)
    area_b = (px2 - px1) * (py2 - py1)
    iou = inter / (area_a + area_b - inter)

    cidx = jax.lax.broadcasted_iota(jnp.int32, (G, N), 1)
    ridx = jax.lax.broadcasted_iota(jnp.int32, (G, N), 0)

    gbo = jnp.max(iou, axis=1, keepdims=True)
    gbi = jnp.min(jnp.where(iou == gbo, cidx, N), axis=1, keepdims=True)
    valid = gbo >= 0.2
    hv = jnp.max(valid.astype(jnp.float32), axis=0, keepdims=True)

    dbo = jnp.max(iou, axis=0, keepdims=True)

    pow2_row = jax.lax.bitcast_convert_type(
        (jax.lax.broadcasted_iota(jnp.int32, (1, G), 1) + 127) << 23,
        jnp.float32)
    pow2_col = jax.lax.bitcast_convert_type(
        (jax.lax.broadcasted_iota(jnp.int32, (G, 1), 0) + 127) << 23,
        jnp.float32)
    dn = (((1,), (0,)), ((), ()))
    hi = jax.lax.Precision.HIGHEST

    mm = gbi == cidx
    eq = iou == dbo
    fbits_f = jax.lax.dot_general(pow2_row, mm.astype(jnp.float32), dn)
    dbits_f = jax.lax.dot_general(pow2_row, eq.astype(jnp.float32), dn)
    fbits = fbits_f.astype(jnp.int32)
    vbits = jnp.sum(pow2_col * valid.astype(jnp.float32), axis=0,
                    keepdims=True).astype(jnp.int32)
    set_mask = (fbits & vbits) != 0
    force = (jax.lax.bitcast_convert_type(fbits_f, jnp.int32) >> 23) - 127
    dbits = dbits_f.astype(jnp.int32)
    low_f = (dbits & (-dbits)).astype(jnp.float32)
    dbi = (jax.lax.bitcast_convert_type(low_f, jnp.int32) >> 23) - 127
    didx = jnp.where(fbits > 0, force, dbi)
    ovl = jnp.where(set_mask, 1.0, dbo)

    oh = didx == ridx
    d1 = jax.lax.dot_general(tt, oh.astype(jnp.float32), dn,
                             precision=hi)
    mxy1 = d1[0:2]
    mxy2 = d1[2:4]
    conf = d1[4:5]

    conf = jnp.where(ovl < 0.5, 0.0, conf)
    conf = jnp.where(hv > 0.0, conf, 0.0)

    pcxy = db[0:2]
    pwh = db[2:4]
    g_cxy = ((mxy1 + mxy2) / 2 - pcxy) / (0.1 * pwh)
    g_wh = jnp.log((mxy2 - mxy1) / pwh) / 0.2

    enc_ref[0] = jnp.concatenate([g_cxy, g_wh, conf], axis=0)


def _loss_body(enc_ref, lp_ref, cp_ref, lossc_ref, stats_ref):
    C, N = cp_ref.shape[1], cp_ref.shape[2]
    e = enc_ref[0]
    lp = lp_ref[0]
    cp = cp_ref[0]
    conf = e[4:5]
    pos = conf > 0.0

    sl = jnp.where(pos, _smooth_l1(lp - e[0:4]), 0.0)
    lloc = jnp.sum(jnp.sum(sl, axis=0, keepdims=True), axis=1,
                   keepdims=True)

    m = jnp.max(cp, axis=0, keepdims=True)
    ex = jnp.exp(cp - m)
    klass = jax.lax.broadcasted_iota(jnp.int32, (C, N), 0)
    conf_i = conf.astype(jnp.int32)
    ones_row = jnp.ones((1, C), jnp.float32)
    dn = (((1,), (0,)), ((), ()))
    s = jax.lax.dot_general(ones_row, ex, dn)
    lse = jnp.log(s) + m
    gathered = jnp.sum(jnp.where(klass == conf_i, cp, 0.0), axis=0,
                       keepdims=True)
    ce = lse - gathered

    sum_ce_pos = jnp.sum(jnp.where(pos, ce, 0.0), axis=1, keepdims=True)
    npos = jnp.sum(pos.astype(jnp.float32), axis=1, keepdims=True)

    lossc_ref[0] = jnp.maximum(jnp.where(pos, 0.0, ce), 0.0)

    lane = jax.lax.broadcasted_iota(jnp.int32, (1, 128), 1)
    svec = jnp.where(lane == 0, lloc,
                     jnp.where(lane == 1, sum_ce_pos,
                               jnp.where(lane == 2, npos, 0.0)))
    stats_ref[0] = svec


def _select_body(lossc_ref, stats_ref, out0_ref, out1_ref):
    B, N = lossc_ref.shape
    lc = lossc_ref[...]
    st = stats_ref[...]
    npos = st[:, 2:3]
    k = jnp.minimum(3 * npos.astype(jnp.int32), N - 2) + 1

    bits = jax.lax.bitcast_convert_type(lc, jnp.int32)
    t = jnp.zeros((B, 1), jnp.int32)
    for bit in range(30, -1, -1):
        cand = t | (1 << bit)
        cnt = jnp.sum((bits >= cand).astype(jnp.int32), axis=1,
                      keepdims=True)
        t = jnp.where(cnt >= k, cand, t)

    tf = jax.lax.bitcast_convert_type(t, jnp.float32)
    gtm = bits > t
    cnt_gt = jnp.sum(gtm.astype(jnp.int32), axis=1, keepdims=True)
    sum_gt = jnp.sum(jnp.where(gtm, lc, 0.0), axis=1, keepdims=True)
    topk = sum_gt + (k - cnt_gt).astype(jnp.float32) * tf

    loss_conf = (jnp.sum(st[:, 1:2], axis=0, keepdims=True)
                 + jnp.sum(topk, axis=0, keepdims=True))
    loss_loc = jnp.sum(st[:, 0:1], axis=0, keepdims=True)
    nn = jnp.maximum(jnp.sum(npos, axis=0, keepdims=True), 1.0)
    out0_ref[...] = loss_loc / nn
    out1_ref[...] = loss_conf / nn


def kernel(loc_p, conf_p, targets, default_boxes):
    B, N, _ = loc_p.shape
    C = conf_p.shape[2]
    G = targets.shape[1]

    loc_pt = jnp.transpose(loc_p, (0, 2, 1))
    conf_pt = jnp.transpose(conf_p, (0, 2, 1))
    targets_t = jnp.transpose(targets, (0, 2, 1))
    db_t = jnp.transpose(default_boxes, (1, 0))

    enc = pl.pallas_call(
        _match_body,
        grid=(B,),
        in_specs=[
            pl.BlockSpec((1, G, 5), lambda b: (b, 0, 0)),
            pl.BlockSpec((1, 5, G), lambda b: (b, 0, 0)),
            pl.BlockSpec((4, N), lambda b: (0, 0)),
        ],
        out_specs=pl.BlockSpec((1, 5, N), lambda b: (b, 0, 0)),
        out_shape=jax.ShapeDtypeStruct((B, 5, N), jnp.float32),
    )(targets, targets_t, db_t)

    lossc, stats = pl.pallas_call(
        _loss_body,
        grid=(B,),
        in_specs=[
            pl.BlockSpec((1, 5, N), lambda b: (b, 0, 0)),
            pl.BlockSpec((1, 4, N), lambda b: (b, 0, 0)),
            pl.BlockSpec((1, C, N), lambda b: (b, 0, 0)),
        ],
        out_specs=[
            pl.BlockSpec((1, 1, N), lambda b: (b, 0, 0)),
            pl.BlockSpec((1, 1, 128), lambda b: (b, 0, 0)),
        ],
        out_shape=[
            jax.ShapeDtypeStruct((B, 1, N), jnp.float32),
            jax.ShapeDtypeStruct((B, 1, 128), jnp.float32),
        ],
    )(enc, loc_pt, conf_pt)

    out0, out1 = pl.pallas_call(
        _select_body,
        out_shape=[
            jax.ShapeDtypeStruct((1, 1), jnp.float32),
            jax.ShapeDtypeStruct((1, 1), jnp.float32),
        ],
    )(lossc.reshape(B, N), stats.reshape(B, 128))

    return (out0.reshape(()), out1.reshape(()))
